# baseline (speedup 1.0000x reference)
import functools

import jax
import jax.numpy as jnp
from jax import lax
from jax.experimental import pallas as pl
from jax.experimental.pallas import tpu as pltpu
from jax.experimental.pallas import tpu_sc as plsc

_CHUNK = 32


def _sc_body(batch, rows_per_w, n_chunks, n_cores, table_hbm, out_hbm, buf_v):
    wid = lax.axis_index("s") * n_cores + lax.axis_index("c")
    base = wid * rows_per_w
    for c in range(n_chunks):
        lo = base + c * _CHUNK
        pltpu.sync_copy(table_hbm.at[pl.ds(lo, _CHUNK)], buf_v)
        for b in range(batch):
            pltpu.sync_copy(buf_v, out_hbm.at[b, pl.ds(lo, _CHUNK)])


def kernel(x, position_embeddings):
    batch = x.shape[0]
    seq_len = x.shape[1]
    n_rows, dim = position_embeddings.shape
    info = plsc.get_sparse_core_info()
    n_workers = info.num_cores * info.num_subcores
    rows_per_w = seq_len // n_workers
    n_chunks = rows_per_w // _CHUNK
    mesh = plsc.VectorSubcoreMesh(core_axis_name="c", subcore_axis_name="s")
    body = functools.partial(_sc_body, batch, rows_per_w, n_chunks, info.num_cores)
    run = pl.kernel(
        body,
        out_type=jax.ShapeDtypeStruct((batch, seq_len, dim), position_embeddings.dtype),
        mesh=mesh,
        scratch_types=[
            pltpu.VMEM((_CHUNK, dim), jnp.float32),
        ],
    )
    return run(position_embeddings)


# speedup vs baseline: 1.0425x; 1.0425x over previous
import functools

import jax
import jax.numpy as jnp
from jax import lax
from jax.experimental import pallas as pl
from jax.experimental.pallas import tpu as pltpu
from jax.experimental.pallas import tpu_sc as plsc

_CHUNK = 32


def _sc_body(batch, rows_per_w, n_chunks, n_cores,
             table_hbm, out_hbm, buf0, buf1, rsem0, rsem1, wsem0, wsem1):
    wid = lax.axis_index("s") * n_cores + lax.axis_index("c")
    base = wid * rows_per_w
    bufs = (buf0, buf1)
    rsems = (rsem0, rsem1)
    wsems = (wsem0, wsem1)

    reads = [None, None]
    writes = [[], []]
    reads[0] = pltpu.async_copy(table_hbm.at[pl.ds(base, _CHUNK)], bufs[0], rsems[0])
    for c in range(n_chunks):
        cur = c & 1
        nxt = (c + 1) & 1
        if c + 1 < n_chunks:
            for h in writes[nxt]:
                h.wait()
            writes[nxt] = []
            lo_n = base + (c + 1) * _CHUNK
            reads[nxt] = pltpu.async_copy(
                table_hbm.at[pl.ds(lo_n, _CHUNK)], bufs[nxt], rsems[nxt])
        reads[cur].wait()
        lo = base + c * _CHUNK
        for b in range(batch):
            writes[cur].append(
                pltpu.async_copy(bufs[cur], out_hbm.at[b, pl.ds(lo, _CHUNK)], wsems[cur]))
    for lst in writes:
        for h in lst:
            h.wait()


def kernel(x, position_embeddings):
    batch = x.shape[0]
    seq_len = x.shape[1]
    n_rows, dim = position_embeddings.shape
    info = plsc.get_sparse_core_info()
    n_workers = info.num_cores * info.num_subcores
    rows_per_w = seq_len // n_workers
    n_chunks = rows_per_w // _CHUNK
    mesh = plsc.VectorSubcoreMesh(core_axis_name="c", subcore_axis_name="s")
    body = functools.partial(_sc_body, batch, rows_per_w, n_chunks, info.num_cores)
    run = pl.kernel(
        body,
        out_type=jax.ShapeDtypeStruct((batch, seq_len, dim), position_embeddings.dtype),
        mesh=mesh,
        scratch_types=[
            pltpu.VMEM((_CHUNK, dim), jnp.float32),
            pltpu.VMEM((_CHUNK, dim), jnp.float32),
            pltpu.SemaphoreType.DMA,
            pltpu.SemaphoreType.DMA,
            pltpu.SemaphoreType.DMA,
            pltpu.SemaphoreType.DMA,
        ],
    )
    return run(position_embeddings)
